# speedup vs baseline: 2.9728x; 2.9728x over previous
import functools

import jax
import jax.numpy as jnp
from jax import lax
from jax.experimental import pallas as pl
from jax.experimental.pallas import tpu as pltpu
from jax.experimental.pallas import tpu_sc as plsc

_NC = 2
_NS = 16
_NW = _NC * _NS
_CH = 128


@functools.partial(jax.jit, static_argnames=("n_rows", "d"))
def _gather_rows(idx, table, n_rows, d):
    b_per_w = n_rows // _NW
    n_chunk = b_per_w // _CH
    mesh = plsc.VectorSubcoreMesh(core_axis_name="c", subcore_axis_name="s")

    @functools.partial(
        pl.kernel,
        mesh=mesh,
        out_type=jax.ShapeDtypeStruct((n_rows, d), jnp.float32),
        scratch_types=[
            pltpu.VMEM((n_chunk, _CH), jnp.int32),
            pltpu.VMEM((_CH, d), jnp.float32),
            pltpu.SemaphoreType.DMA,
        ],
    )
    def k(idx_hbm, table_hbm, out_hbm, idx_v, rows_v, sem):
        wid = lax.axis_index("s") * _NC + lax.axis_index("c")
        base = wid * b_per_w
        pltpu.sync_copy(idx_hbm.at[wid], idx_v)

        def body(j, carry):
            pltpu.async_copy(table_hbm.at[idx_v.at[j]], rows_v, sem).wait()
            pltpu.sync_copy(rows_v, out_hbm.at[pl.ds(base + j * _CH, _CH)])
            return carry

        lax.fori_loop(0, n_chunk, body, 0)

    return k(idx.reshape(_NW, n_chunk, _CH), table)


def kernel(x, tok_emb):
    b, l = x.shape
    v, d = tok_emb.shape
    out = _gather_rows(x.reshape(-1).astype(jnp.int32), tok_emb, b * l, d)
    return out.reshape(b, l, d)


# speedup vs baseline: 3.3369x; 1.1225x over previous
import functools

import jax
import jax.numpy as jnp
from jax import lax
from jax.experimental import pallas as pl
from jax.experimental.pallas import tpu as pltpu
from jax.experimental.pallas import tpu_sc as plsc

_NC = 2
_NS = 16
_NW = _NC * _NS
_CH = 128


@functools.partial(jax.jit, static_argnames=("n_rows", "d"))
def _gather_rows(idx, table, n_rows, d):
    b_per_w = n_rows // _NW
    n_chunk = b_per_w // _CH
    assert n_chunk % 2 == 0 and n_chunk >= 4
    mesh = plsc.VectorSubcoreMesh(core_axis_name="c", subcore_axis_name="s")

    @functools.partial(
        pl.kernel,
        mesh=mesh,
        out_type=jax.ShapeDtypeStruct((n_rows, d), jnp.float32),
        scratch_types=[
            pltpu.VMEM((n_chunk, _CH), jnp.int32),
            pltpu.VMEM((_CH, d), jnp.float32),
            pltpu.VMEM((_CH, d), jnp.float32),
            pltpu.SemaphoreType.DMA,
            pltpu.SemaphoreType.DMA,
            pltpu.SemaphoreType.DMA,
            pltpu.SemaphoreType.DMA,
        ],
    )
    def k(idx_hbm, table_hbm, out_hbm, idx_v, rows0, rows1, g0, g1, s0, s1):
        wid = lax.axis_index("s") * _NC + lax.axis_index("c")
        base = wid * b_per_w
        pltpu.sync_copy(idx_hbm.at[wid], idx_v)

        rows = (rows0, rows1)
        gsem = (g0, g1)
        ssem = (s0, s1)

        def out_slice(j):
            return out_hbm.at[pl.ds(base + j * _CH, _CH)]

        def g_start(j, b):
            pltpu.async_copy(table_hbm.at[idx_v.at[j]], rows[b], gsem[b])

        def g_wait(j, b):
            pltpu.make_async_copy(
                table_hbm.at[idx_v.at[j]], rows[b], gsem[b]).wait()

        def s_start(j, b):
            pltpu.async_copy(rows[b], out_slice(j), ssem[b])

        def s_wait(j, b):
            pltpu.make_async_copy(rows[b], out_slice(j), ssem[b]).wait()

        g_start(0, 0)
        g_start(1, 1)
        g_wait(0, 0)
        s_start(0, 0)

        def body(g, carry):
            for p in range(2):
                j = 2 * g + p
                b = p
                s_wait(j - 2, b)
                g_start(j, b)
                g_wait(j - 1, 1 - b)
                s_start(j - 1, 1 - b)
            return carry

        lax.fori_loop(1, n_chunk // 2, body, 0)

        g_wait(n_chunk - 1, 1)
        s_start(n_chunk - 1, 1)
        s_wait(n_chunk - 2, 0)
        s_wait(n_chunk - 1, 1)

    return k(idx.reshape(_NW, n_chunk, _CH), table)


def kernel(x, tok_emb):
    b, l = x.shape
    v, d = tok_emb.shape
    out = _gather_rows(x.reshape(-1).astype(jnp.int32), tok_emb, b * l, d)
    return out.reshape(b, l, d)


# speedup vs baseline: 5.8304x; 1.7473x over previous
import functools

import jax
import jax.numpy as jnp
from jax import lax
from jax.experimental import pallas as pl
from jax.experimental.pallas import tpu as pltpu
from jax.experimental.pallas import tpu_sc as plsc

_NC = 2
_NS = 16
_NW = _NC * _NS
_EPC = 2


@functools.partial(jax.jit, static_argnames=("bsz", "l", "d"))
def _gather_rows(idx, table, bsz, l, d):
    e_per_w = bsz // _NW
    n_chunk = e_per_w // _EPC
    ch = _EPC * l
    assert n_chunk % 2 == 0 and n_chunk >= 4 and ch <= 128
    mesh = plsc.VectorSubcoreMesh(core_axis_name="c", subcore_axis_name="s")

    @functools.partial(
        pl.kernel,
        mesh=mesh,
        out_type=jax.ShapeDtypeStruct((bsz, l, d), jnp.float32),
        scratch_types=[
            pltpu.VMEM((n_chunk, ch), jnp.int32),
            pltpu.VMEM((ch, d), jnp.float32),
            pltpu.VMEM((ch, d), jnp.float32),
            pltpu.SemaphoreType.DMA,
            pltpu.SemaphoreType.DMA,
            pltpu.SemaphoreType.DMA,
            pltpu.SemaphoreType.DMA,
        ],
    )
    def k(idx_hbm, table_hbm, out_hbm, idx_v, rows0, rows1, g0, g1, s0, s1):
        wid = lax.axis_index("s") * _NC + lax.axis_index("c")
        ebase = wid * e_per_w
        pltpu.sync_copy(idx_hbm.at[wid], idx_v)

        rows = (rows0, rows1)
        gsem = (g0, g1)
        ssem = (s0, s1)

        def g_start(j, b):
            pltpu.async_copy(table_hbm.at[idx_v.at[j]], rows[b], gsem[b])

        def g_wait(j, b):
            pltpu.make_async_copy(
                table_hbm.at[idx_v.at[j]], rows[b], gsem[b]).wait()

        def s_start(j, b):
            e0 = ebase + j * _EPC
            for t in range(_EPC):
                pltpu.async_copy(
                    rows[b].at[pl.ds(t * l, l)], out_hbm.at[e0 + t], ssem[b])

        def s_wait(j, b):
            e0 = ebase + j * _EPC
            for t in range(_EPC):
                pltpu.make_async_copy(
                    rows[b].at[pl.ds(t * l, l)], out_hbm.at[e0 + t],
                    ssem[b]).wait()

        g_start(0, 0)
        g_start(1, 1)
        g_wait(0, 0)
        s_start(0, 0)

        def body(g, carry):
            for p in range(2):
                j = 2 * g + p
                b = p
                s_wait(j - 2, b)
                g_start(j, b)
                g_wait(j - 1, 1 - b)
                s_start(j - 1, 1 - b)
            return carry

        lax.fori_loop(1, n_chunk // 2, body, 0)

        g_wait(n_chunk - 1, 1)
        s_start(n_chunk - 1, 1)
        s_wait(n_chunk - 2, 0)
        s_wait(n_chunk - 1, 1)

    return k(idx.reshape(_NW, n_chunk, ch), table)


def kernel(x, tok_emb):
    b, l = x.shape
    v, d = tok_emb.shape
    out = _gather_rows(x.reshape(-1).astype(jnp.int32), tok_emb, b, l, d)
    return out


# speedup vs baseline: 10.2734x; 1.7620x over previous
import functools

import jax
import jax.numpy as jnp
from jax import lax
from jax.experimental import pallas as pl
from jax.experimental.pallas import tpu as pltpu
from jax.experimental.pallas import tpu_sc as plsc

_NC = 2
_NS = 16
_NW = _NC * _NS
_CH = 128
_NB = 4


@functools.partial(jax.jit, static_argnames=("n_rows", "d"))
def _gather_rows(idx, table, n_rows, d):
    b_per_w = n_rows // _NW
    n_chunk = b_per_w // _CH
    n_steady = (n_chunk - _NB) // _NB * _NB
    assert n_chunk >= 2 * _NB
    mesh = plsc.VectorSubcoreMesh(core_axis_name="c", subcore_axis_name="s")

    @functools.partial(
        pl.kernel,
        mesh=mesh,
        out_type=jax.ShapeDtypeStruct((n_rows, d), jnp.float32),
        scratch_types=(
            [pltpu.VMEM((n_chunk, _CH), jnp.int32)]
            + [pltpu.VMEM((_CH, d), jnp.float32)] * _NB
            + [pltpu.SemaphoreType.DMA] * (2 * _NB)
        ),
    )
    def k(idx_hbm, table_hbm, out_hbm, idx_v, *bufs):
        rows = bufs[:_NB]
        gsem = bufs[_NB:2 * _NB]
        ssem = bufs[2 * _NB:]
        wid = lax.axis_index("s") * _NC + lax.axis_index("c")
        base = wid * b_per_w
        pltpu.sync_copy(idx_hbm.at[wid], idx_v)

        def out_slice(j):
            return out_hbm.at[pl.ds(base + j * _CH, _CH)]

        def g_start(j, b):
            pltpu.async_copy(table_hbm.at[idx_v.at[j]], rows[b], gsem[b])

        def g_wait(j, b):
            pltpu.make_async_copy(
                table_hbm.at[idx_v.at[j]], rows[b], gsem[b]).wait()

        def s_start(j, b):
            pltpu.async_copy(rows[b], out_slice(j), ssem[b])

        def s_wait(j, b):
            pltpu.make_async_copy(rows[b], out_slice(j), ssem[b]).wait()

        def steady(j, b):
            bb = (b + 1) % _NB
            g_wait(j - _NB + 1, bb)
            s_start(j - _NB + 1, bb)
            s_wait(j - _NB, b)
            g_start(j, b)

        for b in range(_NB):
            g_start(b, b)
        g_wait(0, 0)
        s_start(0, 0)

        def body(g, carry):
            for p in range(_NB):
                steady(_NB * g + p, p)
            return carry

        lax.fori_loop(1, 1 + n_steady // _NB, body, 0)

        for j in range(_NB + n_steady, n_chunk):
            steady(j, j % _NB)
        for r in range(_NB - 1):
            jr = n_chunk - _NB + 1 + r
            g_wait(jr, jr % _NB)
            s_start(jr, jr % _NB)
            s_wait(jr - 1, (jr - 1) % _NB)
        s_wait(n_chunk - 1, (n_chunk - 1) % _NB)

    return k(idx.reshape(_NW, n_chunk, _CH), table)


def kernel(x, tok_emb):
    b, l = x.shape
    v, d = tok_emb.shape
    idx_lmajor = x.T.reshape(-1).astype(jnp.int32)
    out = _gather_rows(idx_lmajor, tok_emb, b * l, d)
    return out.reshape(l, b, d).transpose(1, 0, 2)


# speedup vs baseline: 10.3526x; 1.0077x over previous
import functools

import jax
import jax.numpy as jnp
from jax import lax
from jax.experimental import pallas as pl
from jax.experimental.pallas import tpu as pltpu
from jax.experimental.pallas import tpu_sc as plsc

_NC = 2
_NS = 16
_NW = _NC * _NS
_CH = 128
_NB = 4


@functools.partial(jax.jit, static_argnames=("n_rows", "d"))
def _gather_rows(idx, table, n_rows, d):
    b_per_w = n_rows // _NW
    n_chunk = b_per_w // _CH
    n_steady = (n_chunk - _NB) // _NB * _NB
    assert n_chunk >= 2 * _NB
    mesh = plsc.VectorSubcoreMesh(core_axis_name="c", subcore_axis_name="s")

    @functools.partial(
        pl.kernel,
        mesh=mesh,
        out_type=jax.ShapeDtypeStruct((n_rows, d), jnp.float32),
        scratch_types=(
            [pltpu.VMEM((n_chunk, _CH), jnp.int32)]
            + [pltpu.VMEM((_CH, d), jnp.float32)] * _NB
            + [pltpu.SemaphoreType.DMA] * (2 * _NB)
        ),
        compiler_params=pltpu.CompilerParams(skip_device_barrier=True),
    )
    def k(idx_hbm, table_hbm, out_hbm, idx_v, *bufs):
        rows = bufs[:_NB]
        gsem = bufs[_NB:2 * _NB]
        ssem = bufs[2 * _NB:]
        wid = lax.axis_index("s") * _NC + lax.axis_index("c")
        base = wid * b_per_w
        pltpu.sync_copy(idx_hbm.at[wid], idx_v)

        def out_slice(j):
            return out_hbm.at[pl.ds(base + j * _CH, _CH)]

        def g_start(j, b):
            pltpu.async_copy(table_hbm.at[idx_v.at[j]], rows[b], gsem[b])

        def g_wait(j, b):
            pltpu.make_async_copy(
                table_hbm.at[idx_v.at[j]], rows[b], gsem[b]).wait()

        def s_start(j, b):
            pltpu.async_copy(rows[b], out_slice(j), ssem[b])

        def s_wait(j, b):
            pltpu.make_async_copy(rows[b], out_slice(j), ssem[b]).wait()

        def steady(j, b):
            bb = (b + 1) % _NB
            g_wait(j - _NB + 1, bb)
            s_start(j - _NB + 1, bb)
            s_wait(j - _NB, b)
            g_start(j, b)

        for b in range(_NB):
            g_start(b, b)
        g_wait(0, 0)
        s_start(0, 0)

        def body(g, carry):
            for p in range(_NB):
                steady(_NB * g + p, p)
            return carry

        lax.fori_loop(1, 1 + n_steady // _NB, body, 0)

        for j in range(_NB + n_steady, n_chunk):
            steady(j, j % _NB)
        for r in range(_NB - 1):
            jr = n_chunk - _NB + 1 + r
            g_wait(jr, jr % _NB)
            s_start(jr, jr % _NB)
            s_wait(jr - 1, (jr - 1) % _NB)
        s_wait(n_chunk - 1, (n_chunk - 1) % _NB)

    return k(idx.reshape(_NW, n_chunk, _CH), table)


def kernel(x, tok_emb):
    b, l = x.shape
    v, d = tok_emb.shape
    idx_lmajor = x.T.reshape(-1).astype(jnp.int32)
    out = _gather_rows(idx_lmajor, tok_emb, b * l, d)
    return out.reshape(l, b, d).transpose(1, 0, 2)


# speedup vs baseline: 10.6718x; 1.0308x over previous
import functools

import jax
import jax.numpy as jnp
from jax import lax
from jax.experimental import pallas as pl
from jax.experimental.pallas import tpu as pltpu
from jax.experimental.pallas import tpu_sc as plsc

_NC = 2
_NS = 16
_NW = _NC * _NS
_CH = 128
_NB = 4


@functools.partial(jax.jit, static_argnames=("n_rows", "d"))
def _gather_rows(idx_t, table, n_rows, d):
    b_per_w = n_rows // _NW
    n_chunk = b_per_w // _CH
    n_steady = (n_chunk - _NB) // _NB * _NB
    assert n_chunk >= 2 * _NB
    assert idx_t.shape == (n_chunk, _NW * _CH)
    mesh = plsc.VectorSubcoreMesh(core_axis_name="c", subcore_axis_name="s")

    @functools.partial(
        pl.kernel,
        mesh=mesh,
        out_type=jax.ShapeDtypeStruct((n_rows, d), jnp.float32),
        scratch_types=(
            [pltpu.VMEM((n_chunk, _CH), jnp.int32)]
            + [pltpu.VMEM((_CH, d), jnp.float32)] * _NB
            + [pltpu.SemaphoreType.DMA] * (2 * _NB)
        ),
    )
    def k(idx_hbm, table_hbm, out_hbm, idx_v, *bufs):
        rows = bufs[:_NB]
        gsem = bufs[_NB:2 * _NB]
        ssem = bufs[2 * _NB:]
        wid = lax.axis_index("s") * _NC + lax.axis_index("c")
        pltpu.sync_copy(idx_hbm.at[:, pl.ds(wid * _CH, _CH)], idx_v)

        def out_slice(j):
            return out_hbm.at[pl.ds(j * (n_rows // n_chunk) + wid * _CH, _CH)]

        def g_start(j, b):
            pltpu.async_copy(table_hbm.at[idx_v.at[j]], rows[b], gsem[b])

        def g_wait(j, b):
            pltpu.make_async_copy(
                table_hbm.at[idx_v.at[j]], rows[b], gsem[b]).wait()

        def s_start(j, b):
            pltpu.async_copy(rows[b], out_slice(j), ssem[b])

        def s_wait(j, b):
            pltpu.make_async_copy(rows[b], out_slice(j), ssem[b]).wait()

        def steady(j, b):
            bb = (b + 1) % _NB
            g_wait(j - _NB + 1, bb)
            s_start(j - _NB + 1, bb)
            s_wait(j - _NB, b)
            g_start(j, b)

        for b in range(_NB):
            g_start(b, b)
        g_wait(0, 0)
        s_start(0, 0)

        def body(g, carry):
            for p in range(_NB):
                steady(_NB * g + p, p)
            return carry

        lax.fori_loop(1, 1 + n_steady // _NB, body, 0)

        for j in range(_NB + n_steady, n_chunk):
            steady(j, j % _NB)
        for r in range(_NB - 1):
            jr = n_chunk - _NB + 1 + r
            g_wait(jr, jr % _NB)
            s_start(jr, jr % _NB)
            s_wait(jr - 1, (jr - 1) % _NB)
        s_wait(n_chunk - 1, (n_chunk - 1) % _NB)

    return k(idx_t, table)


def kernel(x, tok_emb):
    b, l = x.shape
    v, d = tok_emb.shape
    out = _gather_rows(x.T.astype(jnp.int32), tok_emb, b * l, d)
    return out.reshape(l, b, d).transpose(1, 0, 2)


# speedup vs baseline: 10.7077x; 1.0034x over previous
import functools

import jax
import jax.numpy as jnp
from jax import lax
from jax.experimental import pallas as pl
from jax.experimental.pallas import tpu as pltpu
from jax.experimental.pallas import tpu_sc as plsc

_NC = 2
_NS = 16
_NW = _NC * _NS
_CH = 128
_NB = 4


@functools.partial(jax.jit, static_argnames=("n_rows", "d"))
def _gather_rows(idx_t, table, n_rows, d):
    b_per_w = n_rows // _NW
    n_chunk = b_per_w // _CH
    n_steady = (n_chunk - _NB) // _NB * _NB
    assert n_chunk >= 2 * _NB
    assert idx_t.shape == (n_chunk, _NW * _CH)
    mesh = plsc.VectorSubcoreMesh(core_axis_name="c", subcore_axis_name="s")

    @functools.partial(
        pl.kernel,
        mesh=mesh,
        out_type=jax.ShapeDtypeStruct((n_rows, d), jnp.float32),
        scratch_types=(
            [pltpu.VMEM((n_chunk, _CH), jnp.int32)]
            + [pltpu.VMEM((_CH, d), jnp.float32)] * _NB
            + [pltpu.SemaphoreType.DMA] * (2 * _NB + 1)
        ),
    )
    def k(idx_hbm, table_hbm, out_hbm, idx_v, *bufs):
        rows = bufs[:_NB]
        gsem = bufs[_NB:2 * _NB]
        ssem = bufs[2 * _NB:3 * _NB]
        isem = bufs[3 * _NB]
        wid = lax.axis_index("s") * _NC + lax.axis_index("c")

        def i_copy(j):
            return pltpu.make_async_copy(
                idx_hbm.at[j, pl.ds(wid * _CH, _CH)], idx_v.at[j], isem)

        def i_fire(j, carry):
            i_copy(j).start()
            return carry

        lax.fori_loop(0, n_chunk, i_fire, 0)

        def out_slice(j):
            return out_hbm.at[pl.ds(j * (n_rows // n_chunk) + wid * _CH, _CH)]

        def g_start(j, b):
            i_copy(j).wait()
            pltpu.async_copy(table_hbm.at[idx_v.at[j]], rows[b], gsem[b])

        def g_wait(j, b):
            pltpu.make_async_copy(
                table_hbm.at[idx_v.at[j]], rows[b], gsem[b]).wait()

        def s_start(j, b):
            pltpu.async_copy(rows[b], out_slice(j), ssem[b])

        def s_wait(j, b):
            pltpu.make_async_copy(rows[b], out_slice(j), ssem[b]).wait()

        def steady(j, b):
            bb = (b + 1) % _NB
            g_wait(j - _NB + 1, bb)
            s_start(j - _NB + 1, bb)
            s_wait(j - _NB, b)
            g_start(j, b)

        for b in range(_NB):
            g_start(b, b)
        g_wait(0, 0)
        s_start(0, 0)

        def body(g, carry):
            for p in range(_NB):
                steady(_NB * g + p, p)
            return carry

        lax.fori_loop(1, 1 + n_steady // _NB, body, 0)

        for j in range(_NB + n_steady, n_chunk):
            steady(j, j % _NB)
        for r in range(_NB - 1):
            jr = n_chunk - _NB + 1 + r
            g_wait(jr, jr % _NB)
            s_start(jr, jr % _NB)
            s_wait(jr - 1, (jr - 1) % _NB)
        s_wait(n_chunk - 1, (n_chunk - 1) % _NB)

    return k(idx_t, table)


def kernel(x, tok_emb):
    b, l = x.shape
    v, d = tok_emb.shape
    out = _gather_rows(x.T.astype(jnp.int32), tok_emb, b * l, d)
    return out.reshape(l, b, d).transpose(1, 0, 2)
